# baseline (speedup 1.0000x reference)
import functools

import jax
import jax.numpy as jnp
from jax import lax
from jax.experimental import pallas as pl
from jax.experimental.pallas import tpu as pltpu
from jax.experimental.pallas import tpu_sc as plsc

VOCAB_ROWS = 100000
DIM = 300
DIM_PAD = 304
NUM_CORES = 2
NUM_SUBCORES = 16
NUM_WORKERS = NUM_CORES * NUM_SUBCORES
CHUNK = 128


@functools.partial(jax.jit, static_argnames=("total_rows",))
def _gather_rows(tokens_flat, table, *, total_rows):
    rows_per_worker = total_rows // NUM_WORKERS
    n_chunks = rows_per_worker // CHUNK
    mesh = plsc.VectorSubcoreMesh(core_axis_name="c", subcore_axis_name="s")

    @functools.partial(
        pl.kernel,
        mesh=mesh,
        out_type=jax.ShapeDtypeStruct((total_rows, DIM_PAD), jnp.float32),
        compiler_params=pltpu.CompilerParams(use_tc_tiling_on_sc=False),
        scratch_types=[
            pltpu.VMEM((CHUNK,), jnp.int32),
            pltpu.VMEM((CHUNK, DIM_PAD), jnp.float32),
            pltpu.SemaphoreType.DMA,
        ],
    )
    def body(tokens_hbm, table_hbm, out_hbm, idx_v, rows_v, sem):
        wid = lax.axis_index("s") * NUM_CORES + lax.axis_index("c")
        base = wid * rows_per_worker

        def chunk_step(i, carry):
            off = base + i * CHUNK
            pltpu.sync_copy(tokens_hbm.at[pl.ds(off, CHUNK)], idx_v)
            pltpu.async_copy(table_hbm.at[idx_v], rows_v, sem).wait()
            pltpu.sync_copy(rows_v, out_hbm.at[pl.ds(off, CHUNK)])
            return carry

        lax.fori_loop(0, n_chunks, chunk_step, 0)

    return body(tokens_flat, table)


def kernel(tokens, table):
    batch, seq = tokens.shape
    total_rows = batch * seq
    table_pad = jnp.pad(table, ((0, 0), (0, DIM_PAD - DIM)))
    out = _gather_rows(tokens.reshape(total_rows), table_pad, total_rows=total_rows)
    return out[:, :DIM].reshape(batch, seq, DIM)


# speedup vs baseline: 1.3035x; 1.3035x over previous
import functools

import jax
import jax.numpy as jnp
from jax import lax
from jax.experimental import pallas as pl
from jax.experimental.pallas import tpu as pltpu
from jax.experimental.pallas import tpu_sc as plsc

VOCAB_ROWS = 100000
DIM = 300
DIM_PAD = 384
TAIL = DIM - 256
NUM_CORES = 2
NUM_SUBCORES = 16
NUM_WORKERS = NUM_CORES * NUM_SUBCORES
CHUNK = 64


@functools.partial(jax.jit, static_argnames=("total_rows",))
def _gather_rows(tokens_flat, table_pad, *, total_rows):
    rows_per_worker = total_rows // NUM_WORKERS
    n_chunks = rows_per_worker // CHUNK
    mesh = plsc.VectorSubcoreMesh(core_axis_name="c", subcore_axis_name="s")

    @functools.partial(
        pl.kernel,
        mesh=mesh,
        out_type=jax.ShapeDtypeStruct((total_rows, DIM), jnp.float32),
        compiler_params=pltpu.CompilerParams(
            use_tc_tiling_on_sc=True, needs_layout_passes=False),
        scratch_types=[
            pltpu.VMEM((CHUNK,), jnp.int32),
            pltpu.VMEM((CHUNK, DIM), jnp.float32),
            pltpu.VMEM((CHUNK, 128), jnp.float32),
            pltpu.SemaphoreType.DMA,
        ],
    )
    def body(tokens_hbm, table_hbm, out_hbm, idx_v, rows_v, tail_v, sem):
        wid = lax.axis_index("s") * NUM_CORES + lax.axis_index("c")
        base = wid * rows_per_worker

        def chunk_step(i, carry):
            off = base + i * CHUNK
            pltpu.sync_copy(tokens_hbm.at[pl.ds(off, CHUNK)], idx_v)
            pltpu.async_copy(
                table_hbm.at[:, pl.ds(0, 128)].at[idx_v],
                rows_v.at[:, pl.ds(0, 128)], sem)
            pltpu.async_copy(
                table_hbm.at[:, pl.ds(128, 128)].at[idx_v],
                rows_v.at[:, pl.ds(128, 128)], sem)
            cp = pltpu.async_copy(table_hbm.at[:, pl.ds(256, 128)].at[idx_v],
                                  tail_v, sem)
            cp.wait()
            cp.wait()
            cp.wait()

            def row_fix(j, c):
                for k in range(2):
                    rows_v[j, pl.ds(256 + 16 * k, 16)] = tail_v[j, pl.ds(16 * k, 16)]
                v = tail_v[j, pl.ds(32, 16)]
                col = lax.iota(jnp.int32, 16) + (256 + 32)
                msk = col < DIM
                plsc.store_scatter(
                    rows_v, [jnp.full((16,), j, jnp.int32), col], v, mask=msk)
                return c

            lax.fori_loop(0, CHUNK, row_fix, 0)
            pltpu.sync_copy(rows_v, out_hbm.at[pl.ds(off, CHUNK)])
            return carry

        lax.fori_loop(0, n_chunks, chunk_step, 0)

    return body(tokens_flat, table_pad)


def kernel(tokens, table):
    batch, seq = tokens.shape
    total_rows = batch * seq
    table_pad = jnp.pad(table, ((0, 0), (0, DIM_PAD - DIM)))
    out = _gather_rows(tokens.reshape(total_rows), table_pad, total_rows=total_rows)
    return out.reshape(batch, seq, DIM)


# speedup vs baseline: 2.0345x; 1.5608x over previous
import functools

import jax
import jax.numpy as jnp
from jax import lax
from jax.experimental import pallas as pl
from jax.experimental.pallas import tpu as pltpu
from jax.experimental.pallas import tpu_sc as plsc

VOCAB_ROWS = 100000
DIM = 300
TAIL_AT = 256
NUM_CORES = 2
NUM_SUBCORES = 16
NUM_WORKERS = NUM_CORES * NUM_SUBCORES
CHUNK = 80


@functools.partial(jax.jit, static_argnames=("total_rows",))
def _gather_rows(tokens_flat, table, tail_tab, *, total_rows):
    rows_per_worker = total_rows // NUM_WORKERS
    n_chunks = rows_per_worker // CHUNK
    mesh = plsc.VectorSubcoreMesh(core_axis_name="c", subcore_axis_name="s")

    @functools.partial(
        pl.kernel,
        mesh=mesh,
        out_type=jax.ShapeDtypeStruct((total_rows, DIM), jnp.float32),
        compiler_params=pltpu.CompilerParams(
            use_tc_tiling_on_sc=True, needs_layout_passes=False),
        scratch_types=[
            pltpu.VMEM((2, CHUNK), jnp.int32),
            pltpu.VMEM((2, CHUNK, DIM), jnp.float32),
            pltpu.VMEM((2, CHUNK, 128), jnp.float32),
            pltpu.SemaphoreType.DMA,
            pltpu.SemaphoreType.DMA,
            pltpu.SemaphoreType.DMA,
            pltpu.SemaphoreType.DMA,
            pltpu.SemaphoreType.DMA,
        ],
    )
    def body(tokens_hbm, table_hbm, tail_hbm, out_hbm, idx_v, rows_v, tail_v,
             idx_sem, g_sem0, g_sem1, st_sem0, st_sem1):
        wid = lax.axis_index("s") * NUM_CORES + lax.axis_index("c")
        base = wid * rows_per_worker
        g_sems = (g_sem0, g_sem1)
        st_sems = (st_sem0, st_sem1)

        def load_idx(g, slot):
            off = base + g * CHUNK
            pltpu.async_copy(tokens_hbm.at[pl.ds(off, CHUNK)],
                             idx_v.at[slot], idx_sem)

        def start_gathers(slot, gs):
            idx = idx_v.at[slot]
            pltpu.async_copy(table_hbm.at[:, pl.ds(0, 128)].at[idx],
                             rows_v.at[slot].at[:, pl.ds(0, 128)], gs)
            pltpu.async_copy(table_hbm.at[:, pl.ds(128, 128)].at[idx],
                             rows_v.at[slot].at[:, pl.ds(128, 128)], gs)
            pltpu.async_copy(tail_hbm.at[idx], tail_v.at[slot], gs)

        def wait_gathers(slot, gs):
            cp = pltpu.make_async_copy(tail_hbm.at[idx_v.at[slot]],
                                       tail_v.at[slot], gs)
            cp.wait()
            cp.wait()
            cp.wait()

        def fixup(slot):
            def row_fix(j, c):
                for k in range(2):
                    rows_v[slot, j, pl.ds(TAIL_AT + 16 * k, 16)] = (
                        tail_v[slot, j, pl.ds(16 * k, 16)])
                v = tail_v[slot, j, pl.ds(32, 16)]
                col = lax.iota(jnp.int32, 16) + (TAIL_AT + 32)
                msk = col < DIM
                plsc.store_scatter(
                    rows_v.at[slot],
                    [jnp.full((16,), j, jnp.int32), col], v, mask=msk)
                return c

            lax.fori_loop(0, CHUNK, row_fix, 0)

        def start_store(g, slot, ss):
            off = base + g * CHUNK
            pltpu.async_copy(rows_v.at[slot], out_hbm.at[pl.ds(off, CHUNK)], ss)

        def wait_store(g, slot, ss):
            off = base + g * CHUNK
            pltpu.make_async_copy(rows_v.at[slot],
                                  out_hbm.at[pl.ds(off, CHUNK)], ss).wait()

        load_idx(0, 0)
        pltpu.make_async_copy(tokens_hbm.at[pl.ds(base, CHUNK)],
                              idx_v.at[0], idx_sem).wait()
        start_gathers(0, g_sems[0])
        load_idx(1, 1)

        def step(g, carry):
            p = lax.rem(g, 2)

            def with_slot(p_static):
                q = 1 - p_static
                @pl.when(g < n_chunks - 1)
                def _():
                    pltpu.make_async_copy(
                        tokens_hbm.at[pl.ds(base, CHUNK)],
                        idx_v.at[q], idx_sem).wait()

                    @pl.when(g >= 1)
                    def _():
                        wait_store(g - 1, q, st_sems[q])

                    start_gathers(q, g_sems[q])

                wait_gathers(p_static, g_sems[p_static])

                @pl.when(g < n_chunks - 2)
                def _():
                    load_idx(g + 2, p_static)

                fixup(p_static)
                start_store(g, p_static, st_sems[p_static])

            lax.cond(p == 0, lambda: with_slot(0), lambda: with_slot(1))
            return carry

        lax.fori_loop(0, n_chunks, step, 0)
        wait_store(n_chunks - 2, 0, st_sems[0])
        wait_store(n_chunks - 1, 1, st_sems[1])

    return body(tokens_flat, table, tail_tab)


def kernel(tokens, table):
    batch, seq = tokens.shape
    total_rows = batch * seq
    tail_tab = jnp.pad(table[:, TAIL_AT:], ((0, 0), (0, 128 - (DIM - TAIL_AT))))
    out = _gather_rows(tokens.reshape(total_rows), table, tail_tab,
                       total_rows=total_rows)
    return out.reshape(batch, seq, DIM)


# speedup vs baseline: 2.0414x; 1.0034x over previous
import functools

import jax
import jax.numpy as jnp
from jax import lax
from jax.experimental import pallas as pl
from jax.experimental.pallas import tpu as pltpu
from jax.experimental.pallas import tpu_sc as plsc

VOCAB_ROWS = 100000
DIM = 300
TAIL_AT = 256
NUM_CORES = 2
NUM_SUBCORES = 16
NUM_WORKERS = NUM_CORES * NUM_SUBCORES
CHUNK = 80


@functools.partial(jax.jit, static_argnames=("total_rows",))
def _gather_rows(tokens_flat, table, tail_tab, *, total_rows):
    rows_per_worker = total_rows // NUM_WORKERS
    n_chunks = rows_per_worker // CHUNK
    mesh = plsc.VectorSubcoreMesh(core_axis_name="c", subcore_axis_name="s")

    @functools.partial(
        pl.kernel,
        mesh=mesh,
        out_type=jax.ShapeDtypeStruct((total_rows, DIM), jnp.float32),
        compiler_params=pltpu.CompilerParams(
            use_tc_tiling_on_sc=True, needs_layout_passes=False),
        scratch_types=[
            pltpu.VMEM((3, CHUNK), jnp.int32),
            pltpu.VMEM((3, CHUNK, DIM), jnp.float32),
            pltpu.VMEM((3, CHUNK, 128), jnp.float32),
            pltpu.SemaphoreType.DMA,
            pltpu.SemaphoreType.DMA,
            pltpu.SemaphoreType.DMA,
            pltpu.SemaphoreType.DMA,
            pltpu.SemaphoreType.DMA,
            pltpu.SemaphoreType.DMA,
            pltpu.SemaphoreType.DMA,
        ],
    )
    def body(tokens_hbm, table_hbm, tail_hbm, out_hbm, idx_v, rows_v, tail_v,
             idx_sem, g_sem0, g_sem1, g_sem2, st_sem0, st_sem1, st_sem2):
        wid = lax.axis_index("s") * NUM_CORES + lax.axis_index("c")
        base = wid * rows_per_worker
        g_sems = (g_sem0, g_sem1, g_sem2)
        st_sems = (st_sem0, st_sem1, st_sem2)

        def load_idx(g, slot):
            off = base + g * CHUNK
            pltpu.async_copy(tokens_hbm.at[pl.ds(off, CHUNK)],
                             idx_v.at[slot], idx_sem)

        def start_gathers(slot, gs):
            idx = idx_v.at[slot]
            pltpu.async_copy(table_hbm.at[:, pl.ds(0, 128)].at[idx],
                             rows_v.at[slot].at[:, pl.ds(0, 128)], gs)
            pltpu.async_copy(table_hbm.at[:, pl.ds(128, 128)].at[idx],
                             rows_v.at[slot].at[:, pl.ds(128, 128)], gs)
            pltpu.async_copy(tail_hbm.at[idx], tail_v.at[slot], gs)

        def wait_gathers(slot, gs):
            cp = pltpu.make_async_copy(tail_hbm.at[idx_v.at[slot]],
                                       tail_v.at[slot], gs)
            cp.wait()
            cp.wait()
            cp.wait()

        def fixup(slot):
            col = lax.iota(jnp.int32, 16) + (TAIL_AT + 32)
            msk = col < DIM

            def row_fix(j, c):
                for k in range(2):
                    rows_v[slot, j, pl.ds(TAIL_AT + 16 * k, 16)] = (
                        tail_v[slot, j, pl.ds(16 * k, 16)])
                v = tail_v[slot, j, pl.ds(32, 16)]
                plsc.store_scatter(
                    rows_v.at[slot],
                    [jnp.full((16,), j, jnp.int32), col], v, mask=msk)
                return c

            lax.fori_loop(0, CHUNK, row_fix, 0)

        def start_store(g, slot, ss):
            off = base + g * CHUNK
            pltpu.async_copy(rows_v.at[slot], out_hbm.at[pl.ds(off, CHUNK)], ss)

        def wait_store(g, slot, ss):
            off = base + g * CHUNK
            pltpu.make_async_copy(rows_v.at[slot],
                                  out_hbm.at[pl.ds(off, CHUNK)], ss).wait()

        load_idx(0, 0)
        pltpu.make_async_copy(tokens_hbm.at[pl.ds(base, CHUNK)],
                              idx_v.at[0], idx_sem).wait()
        start_gathers(0, g_sems[0])
        load_idx(1, 1)

        def step(g, carry):
            p = lax.rem(g, 3)

            def with_slot(p_static):
                q = (p_static + 1) % 3
                r = (p_static + 2) % 3
                @pl.when(g < n_chunks - 1)
                def _():
                    pltpu.make_async_copy(
                        tokens_hbm.at[pl.ds(base, CHUNK)],
                        idx_v.at[q], idx_sem).wait()

                    @pl.when(g >= 2)
                    def _():
                        wait_store(g - 2, q, st_sems[q])

                    start_gathers(q, g_sems[q])

                @pl.when(g < n_chunks - 2)
                def _():
                    load_idx(g + 2, r)

                wait_gathers(p_static, g_sems[p_static])
                fixup(p_static)
                start_store(g, p_static, st_sems[p_static])

            lax.switch(p, [lambda: with_slot(0), lambda: with_slot(1),
                           lambda: with_slot(2)])
            return carry

        lax.fori_loop(0, n_chunks, step, 0)
        wait_store(n_chunks - 3, (n_chunks - 3) % 3,
                   st_sems[(n_chunks - 3) % 3])
        wait_store(n_chunks - 2, (n_chunks - 2) % 3,
                   st_sems[(n_chunks - 2) % 3])
        wait_store(n_chunks - 1, (n_chunks - 1) % 3,
                   st_sems[(n_chunks - 1) % 3])

    return body(tokens_flat, table, tail_tab)


def kernel(tokens, table):
    batch, seq = tokens.shape
    total_rows = batch * seq
    tail_tab = jnp.pad(table[:, TAIL_AT:], ((0, 0), (0, 128 - (DIM - TAIL_AT))))
    out = _gather_rows(tokens.reshape(total_rows), table, tail_tab,
                       total_rows=total_rows)
    return out.reshape(batch, seq, DIM)
